# baseline (speedup 1.0000x reference)
import functools

import jax
import jax.numpy as jnp
from jax import lax
from jax.experimental import pallas as pl
from jax.experimental.pallas import tpu as pltpu
from jax.experimental.pallas import tpu_sc as plsc

N = 16384
HID = 1024
D = 256
P = HID // D
L = 16
NC, NS = 2, 16
NW = NC * NS
ROWS_W = N // NW
CF = 8
C = CF * P
NCH = ROWS_W // CF
NBUF = 6
AHEAD = 3
TAB_PAD = 512
W = 128


def _sc_embed_add(x, idx_all, tab_all):
    mesh = plsc.VectorSubcoreMesh(core_axis_name="c", subcore_axis_name="s")

    @functools.partial(
        pl.kernel,
        mesh=mesh,
        out_type=jax.ShapeDtypeStruct((N, HID), jnp.float32),
        scratch_types=(
            [pltpu.VMEM((ROWS_W * P,), jnp.int32)]
            + [pltpu.VMEM((CF, HID), jnp.float32) for _ in range(NBUF)]
            + [pltpu.VMEM((C, W), jnp.int32) for _ in range(NBUF)]
            + [pltpu.SemaphoreType.DMA for _ in range(NBUF)]
            + [pltpu.SemaphoreType.DMA for _ in range(NBUF)]
            + [pltpu.SemaphoreType.DMA for _ in range(NBUF)]
        ),
    )
    def k(x_hbm, idx_hbm, tab_hbm, out_hbm, idx_v, *bufs_sems):
        xb = bufs_sems[0:NBUF]
        gb = bufs_sems[NBUF:2 * NBUF]
        sx = bufs_sems[2 * NBUF:3 * NBUF]
        sg = bufs_sems[3 * NBUF:4 * NBUF]
        so = bufs_sems[4 * NBUF:5 * NBUF]
        sid = lax.axis_index("s")
        wid = sid * NC + lax.axis_index("c")
        rbase = wid * ROWS_W
        fbase = rbase * P

        pltpu.sync_copy(idx_hbm.at[pl.ds(fbase, ROWS_W * P)], idx_v)

        def issue_in(ci, b):
            pltpu.async_copy(
                x_hbm.at[pl.ds(rbase + ci * CF, CF)], xb[b], sx[b])
            pltpu.async_copy(
                tab_hbm.at[idx_v.at[pl.ds(ci * C, C)]], gb[b], sg[b])

        def wait_in(b):
            pltpu.make_async_copy(x_hbm.at[pl.ds(0, CF)], xb[b], sx[b]).wait()
            pltpu.make_async_copy(
                tab_hbm.at[idx_v.at[pl.ds(0, C)]], gb[b], sg[b]).wait()

        def accumulate(b):
            def row_body(fr, _):
                g0 = fr * P
                for q in range(P):
                    for m in range(W // L):
                        w = gb[b][g0 + q, pl.ds(m * L, L)]
                        lo = lax.bitcast_convert_type(
                            lax.shift_left(w, jnp.int32(16)), jnp.float32)
                        hi = lax.bitcast_convert_type(
                            lax.bitwise_and(w, jnp.int32(-65536)),
                            jnp.float32)
                        plsc.addupdate(
                            xb[b].at[fr, pl.ds(q * D + 2 * m * L, L)], lo)
                        plsc.addupdate(
                            xb[b].at[fr, pl.ds(q * D + 2 * m * L + L, L)], hi)
                return 0

            lax.fori_loop(0, CF, row_body, 0)

        def issue_out(ci, b):
            pltpu.async_copy(
                xb[b], out_hbm.at[pl.ds(rbase + ci * CF, CF)], so[b])

        def wait_out(b):
            pltpu.make_async_copy(
                xb[b], out_hbm.at[pl.ds(0, CF)], so[b]).wait()

        def step(ci, b):
            ca = ci + AHEAD
            bn = (b + AHEAD) % NBUF
            wait_out(bn)
            issue_in(ca, bn)
            wait_in(b)
            accumulate(b)
            issue_out(ci, b)

        def finish(ci, b):
            wait_in(b)
            accumulate(b)
            issue_out(ci, b)

        n_groups = (NCH - NBUF - AHEAD) // NBUF
        rem = (NCH - NBUF - AHEAD) % NBUF

        for ci in range(AHEAD):
            issue_in(ci, ci)
        for ci in range(NBUF):
            ca = ci + AHEAD
            if ca < NBUF:
                issue_in(ca, ca)
            else:
                wait_out(ca % NBUF)
                issue_in(ca, ca % NBUF)
            wait_in(ci)
            accumulate(ci)
            issue_out(ci, ci)

        def body(k, _):
            g = NBUF * k + NBUF
            for b in range(NBUF):
                step(g + b, b)
            return 0

        lax.fori_loop(0, n_groups, body, 0)
        for ci in range(NBUF + n_groups * NBUF, NCH - AHEAD):
            step(ci, ci % NBUF)
        for ci in range(NCH - AHEAD, NCH):
            finish(ci, ci % NBUF)
        for ci in range(NCH - NBUF, NCH):
            wait_out(ci % NBUF)

    return k(x, idx_all, tab_all)


def kernel(x, chan_ind, spat_ind, embed_channel, embed_spatial):
    c0 = chan_ind[:, 0].astype(jnp.int32)
    c1 = chan_ind[:, 1].astype(jnp.int32)
    s0 = spat_ind[:, 0].astype(jnp.int32)
    s1 = spat_ind[:, 1].astype(jnp.int32)
    n_ch = embed_channel.shape[0]
    idx_all = jnp.stack([c0, c1, s0 + n_ch, s1 + n_ch], axis=1).reshape(-1)
    tab_all = jnp.concatenate(
        [embed_channel.astype(jnp.float32), embed_spatial.astype(jnp.float32),
         jnp.zeros((TAB_PAD - embed_channel.shape[0] - embed_spatial.shape[0],
                    D), jnp.float32)],
        axis=0)
    tab_bf = tab_all.astype(jnp.bfloat16)
    t4 = tab_bf.reshape(TAB_PAD, W // L, 2, L).transpose(0, 1, 3, 2)
    tab_packed = jax.lax.bitcast_convert_type(t4, jnp.int32).reshape(
        TAB_PAD, W)
    return _sc_embed_add(x, idx_all, tab_packed)


# speedup vs baseline: 1.0072x; 1.0072x over previous
import functools

import jax
import jax.numpy as jnp
from jax import lax
from jax.experimental import pallas as pl
from jax.experimental.pallas import tpu as pltpu
from jax.experimental.pallas import tpu_sc as plsc

N = 16384
HID = 1024
D = 256
P = HID // D
L = 16
NC, NS = 2, 16
NW = NC * NS
ROWS_W = N // NW
CF = 8
C = CF * P
NCH = ROWS_W // CF
NBUF = 4
AHEAD = 2
TAB_PAD = 512
W = 128


def _sc_embed_add(x, idx_all, tab_all):
    mesh = plsc.VectorSubcoreMesh(core_axis_name="c", subcore_axis_name="s")

    @functools.partial(
        pl.kernel,
        mesh=mesh,
        out_type=jax.ShapeDtypeStruct((N, HID), jnp.float32),
        scratch_types=(
            [pltpu.VMEM((ROWS_W * P,), jnp.int32)]
            + [pltpu.VMEM((CF, HID), jnp.float32) for _ in range(NBUF)]
            + [pltpu.VMEM((C, W), jnp.int32) for _ in range(NBUF)]
            + [pltpu.SemaphoreType.DMA for _ in range(NBUF)]
            + [pltpu.SemaphoreType.DMA for _ in range(NBUF)]
            + [pltpu.SemaphoreType.DMA for _ in range(NBUF)]
        ),
    )
    def k(x_hbm, idx_hbm, tab_hbm, out_hbm, idx_v, *bufs_sems):
        xb = bufs_sems[0:NBUF]
        gb = bufs_sems[NBUF:2 * NBUF]
        sx = bufs_sems[2 * NBUF:3 * NBUF]
        sg = bufs_sems[3 * NBUF:4 * NBUF]
        so = bufs_sems[4 * NBUF:5 * NBUF]
        sid = lax.axis_index("s")
        wid = sid * NC + lax.axis_index("c")
        rbase = wid * ROWS_W
        fbase = rbase * P

        pltpu.sync_copy(idx_hbm.at[pl.ds(fbase, ROWS_W * P)], idx_v)

        def issue_in(ci, b):
            pltpu.async_copy(
                x_hbm.at[pl.ds(rbase + ci * CF, CF)], xb[b], sx[b])
            pltpu.async_copy(
                tab_hbm.at[idx_v.at[pl.ds(ci * C, C)]], gb[b], sg[b])

        def wait_in(b):
            pltpu.make_async_copy(x_hbm.at[pl.ds(0, CF)], xb[b], sx[b]).wait()
            pltpu.make_async_copy(
                tab_hbm.at[idx_v.at[pl.ds(0, C)]], gb[b], sg[b]).wait()

        def accumulate(b):
            def row_body(fr, _):
                g0 = fr * P
                for q in range(P):
                    for m in range(W // L):
                        w = gb[b][g0 + q, pl.ds(m * L, L)]
                        lo = lax.bitcast_convert_type(
                            lax.shift_left(w, jnp.int32(16)), jnp.float32)
                        hi = lax.bitcast_convert_type(
                            lax.bitwise_and(w, jnp.int32(-65536)),
                            jnp.float32)
                        plsc.addupdate(
                            xb[b].at[fr, pl.ds(q * D + 2 * m * L, L)], lo)
                        plsc.addupdate(
                            xb[b].at[fr, pl.ds(q * D + 2 * m * L + L, L)], hi)
                return 0

            lax.fori_loop(0, CF, row_body, 0)

        def issue_out(ci, b):
            pltpu.async_copy(
                xb[b], out_hbm.at[pl.ds(rbase + ci * CF, CF)], so[b])

        def wait_out(b):
            pltpu.make_async_copy(
                xb[b], out_hbm.at[pl.ds(0, CF)], so[b]).wait()

        def step(ci, b):
            ca = ci + AHEAD
            bn = (b + AHEAD) % NBUF
            wait_out(bn)
            issue_in(ca, bn)
            wait_in(b)
            accumulate(b)
            issue_out(ci, b)

        def finish(ci, b):
            wait_in(b)
            accumulate(b)
            issue_out(ci, b)

        n_groups = (NCH - NBUF - AHEAD) // NBUF
        rem = (NCH - NBUF - AHEAD) % NBUF

        for ci in range(AHEAD):
            issue_in(ci, ci)
        for ci in range(NBUF):
            ca = ci + AHEAD
            if ca < NBUF:
                issue_in(ca, ca)
            else:
                wait_out(ca % NBUF)
                issue_in(ca, ca % NBUF)
            wait_in(ci)
            accumulate(ci)
            issue_out(ci, ci)

        def body(k, _):
            g = NBUF * k + NBUF
            for b in range(NBUF):
                step(g + b, b)
            return 0

        lax.fori_loop(0, n_groups, body, 0)
        for ci in range(NBUF + n_groups * NBUF, NCH - AHEAD):
            step(ci, ci % NBUF)
        for ci in range(NCH - AHEAD, NCH):
            finish(ci, ci % NBUF)
        for ci in range(NCH - NBUF, NCH):
            wait_out(ci % NBUF)

    return k(x, idx_all, tab_all)


def kernel(x, chan_ind, spat_ind, embed_channel, embed_spatial):
    c0 = chan_ind[:, 0].astype(jnp.int32)
    c1 = chan_ind[:, 1].astype(jnp.int32)
    s0 = spat_ind[:, 0].astype(jnp.int32)
    s1 = spat_ind[:, 1].astype(jnp.int32)
    n_ch = embed_channel.shape[0]
    idx_all = jnp.stack([c0, c1, s0 + n_ch, s1 + n_ch], axis=1).reshape(-1)
    tab_all = jnp.concatenate(
        [embed_channel.astype(jnp.float32), embed_spatial.astype(jnp.float32),
         jnp.zeros((TAB_PAD - embed_channel.shape[0] - embed_spatial.shape[0],
                    D), jnp.float32)],
        axis=0)
    tab_bf = tab_all.astype(jnp.bfloat16)
    t4 = tab_bf.reshape(TAB_PAD, W // L, 2, L).transpose(0, 1, 3, 2)
    tab_packed = jax.lax.bitcast_convert_type(t4, jnp.int32).reshape(
        TAB_PAD, W)
    return _sc_embed_add(x, idx_all, tab_packed)
